# baseline (speedup 1.0000x reference)
import functools
import jax
import jax.numpy as jnp
from jax import lax
from jax.experimental import pallas as pl
from jax.experimental.pallas import tpu as pltpu
from jax.experimental.pallas import tpu_sc as plsc

_V = 1000001
_B = 16384
_D = 64
_L = 16
_NC = 2
_NS = 16
_NW = _NC * _NS
_BPW = _B // _NW
_NCB = 7812
_VMAIN = _NCB * 128

_mesh = plsc.VectorSubcoreMesh(core_axis_name="c", subcore_axis_name="s")


@functools.partial(
    pl.kernel,
    out_type=jax.ShapeDtypeStruct((_D, _B), jnp.float32),
    mesh=_mesh,
    compiler_params=pltpu.CompilerParams(needs_layout_passes=False),
    scratch_types=[
        pltpu.VMEM((_BPW,), jnp.int32),
        pltpu.VMEM((_BPW,), jnp.int32),
        pltpu.VMEM((_D, 128), jnp.float32),
        pltpu.VMEM((_D, 128), jnp.float32),
        pltpu.VMEM((_D, 128), jnp.float32),
        pltpu.VMEM((_D, 128), jnp.float32),
        pltpu.VMEM((_D, 128), jnp.float32),
        pltpu.VMEM((_D, 128), jnp.float32),
        pltpu.VMEM((_D, 128), jnp.float32),
        pltpu.VMEM((_D, 128), jnp.float32),
        pltpu.VMEM((_D, 128), jnp.float32),
        pltpu.VMEM((_D, 128), jnp.float32),
        pltpu.VMEM((_D, 128), jnp.float32),
        pltpu.VMEM((_D, 128), jnp.float32),
        pltpu.VMEM((_D, 128), jnp.float32),
        pltpu.VMEM((_D, 128), jnp.float32),
        pltpu.VMEM((_D, _BPW // 4), jnp.float32),
        pltpu.SemaphoreType.DMA,
        pltpu.SemaphoreType.DMA,
    ],
)
def _emb_sum(x0_hbm, x1_hbm, w0t_hbm, w1t_hbm, out_hbm,
             idx0_v, idx1_v,
             blk0a_v, blk0b_v, blk0c_v, blk0d_v, blk0e_v, blk0f_v, blk0g_v,
             blk1a_v, blk1b_v, blk1c_v, blk1d_v, blk1e_v, blk1f_v, blk1g_v,
             slab_v, sem0, sem1):
    wid = lax.axis_index("s") * _NC + lax.axis_index("c")
    base = wid * _BPW
    pltpu.sync_copy(x0_hbm.at[pl.ds(base, _BPW)], idx0_v)
    pltpu.sync_copy(x1_hbm.at[pl.ds(base, _BPW)], idx1_v)

    iota = lax.iota(jnp.int32, _L)
    dvs = [16 * c + iota for c in range(_D // _L)]
    blk0 = [blk0a_v, blk0b_v, blk0c_v, blk0d_v, blk0e_v, blk0f_v, blk0g_v]
    blk1 = [blk1a_v, blk1b_v, blk1c_v, blk1d_v, blk1e_v, blk1f_v, blk1g_v]
    _RING = 7

    _G = 2 * _L

    def group(g, carry):
        r0 = g * _G
        vecs0 = [idx0_v[pl.ds(r0, _L)], idx0_v[pl.ds(r0 + _L, _L)]]
        vecs1 = [idx1_v[pl.ds(r0, _L)], idx1_v[pl.ds(r0 + _L, _L)]]
        cbs0 = [jnp.minimum(lax.shift_right_logical(v, 7), _NCB - 1)
                for v in vecs0]
        cbs1 = [jnp.minimum(lax.shift_right_logical(v, 7), _NCB - 1)
                for v in vecs1]
        lanes0 = [v - cb * 128 for v, cb in zip(vecs0, cbs0)]
        lanes1 = [v - cb * 128 for v, cb in zip(vecs1, cbs1)]

        cps = [None] * _RING

        def fire(j):
            p = j % _RING
            q, r = divmod(j, _L)
            cp0 = pltpu.async_copy(
                w0t_hbm.at[:, pl.ds(cbs0[q][r] * 128, 128)], blk0[p], sem0)
            cp1 = pltpu.async_copy(
                w1t_hbm.at[:, pl.ds(cbs1[q][r] * 128, 128)], blk1[p], sem1)
            cps[p] = (cp0, cp1)

        for j in range(_RING - 1):
            fire(j)
        for j in range(_G):
            p = j % _RING
            cp0, cp1 = cps[p]
            cp0.wait()
            cp1.wait()
            if j + _RING - 1 < _G:
                fire(j + _RING - 1)
            q, r = divmod(j, _L)
            l0 = lanes0[q][r]
            l1 = lanes1[q][r]
            f0 = jnp.full((_L,), (l0 < 128).astype(jnp.float32))
            f1 = jnp.full((_L,), (l1 < 128).astype(jnp.float32))
            l0v = jnp.full((_L,), jnp.minimum(l0, 127))
            l1v = jnp.full((_L,), jnp.minimum(l1, 127))
            rv = jnp.full((_L,), lax.rem(g, 4) * _G + j)
            for c in range(_D // _L):
                e0 = plsc.load_gather(blk0[p], [dvs[c], l0v])
                e1 = plsc.load_gather(blk1[p], [dvs[c], l1v])
                plsc.store_scatter(slab_v, [dvs[c], rv], e0 * f0 + e1 * f1)
        @pl.when(lax.rem(g, 4) == 3)
        def _flush():
            quarter = lax.div(g, 4) * (_BPW // 4)
            pltpu.sync_copy(slab_v,
                            out_hbm.at[:, pl.ds(base + quarter, _BPW // 4)])

        return carry

    lax.fori_loop(0, _BPW // _G, group, 0)


def kernel(x, W0, W1):
    x = x.astype(jnp.int32)
    x0 = x[:, 0]
    x1 = x[:, 1]
    out = _emb_sum(x0, x1, W0.T, W1.T).T
    tail0 = jnp.take(W0[_VMAIN:], jnp.clip(x0 - _VMAIN, 0, _V - _VMAIN - 1),
                     axis=0)
    tail1 = jnp.take(W1[_VMAIN:], jnp.clip(x1 - _VMAIN, 0, _V - _VMAIN - 1),
                     axis=0)
    out = out + jnp.where((x0 >= _VMAIN)[:, None], tail0, 0.0)
    out = out + jnp.where((x1 >= _VMAIN)[:, None], tail1, 0.0)
    return out


# speedup vs baseline: 1.0023x; 1.0023x over previous
import functools
import jax
import jax.numpy as jnp
from jax import lax
from jax.experimental import pallas as pl
from jax.experimental.pallas import tpu as pltpu
from jax.experimental.pallas import tpu_sc as plsc

_V = 1000001
_B = 16384
_D = 64
_L = 16
_NC = 2
_NS = 16
_NW = _NC * _NS
_BPW = _B // _NW
_NCB = 7812
_VMAIN = _NCB * 128

_mesh = plsc.VectorSubcoreMesh(core_axis_name="c", subcore_axis_name="s")


@functools.partial(
    pl.kernel,
    out_type=jax.ShapeDtypeStruct((_D, _B), jnp.float32),
    mesh=_mesh,
    compiler_params=pltpu.CompilerParams(needs_layout_passes=False),
    scratch_types=[
        pltpu.VMEM((_BPW,), jnp.int32),
        pltpu.VMEM((_BPW,), jnp.int32),
        pltpu.VMEM((_D, 128), jnp.float32),
        pltpu.VMEM((_D, 128), jnp.float32),
        pltpu.VMEM((_D, 128), jnp.float32),
        pltpu.VMEM((_D, 128), jnp.float32),
        pltpu.VMEM((_D, 128), jnp.float32),
        pltpu.VMEM((_D, 128), jnp.float32),
        pltpu.VMEM((_D, 128), jnp.float32),
        pltpu.VMEM((_D, 128), jnp.float32),
        pltpu.VMEM((_D, 128), jnp.float32),
        pltpu.VMEM((_D, 128), jnp.float32),
        pltpu.VMEM((_D, 128), jnp.float32),
        pltpu.VMEM((_D, 128), jnp.float32),
        pltpu.VMEM((_D, _BPW // 2), jnp.float32),
        pltpu.SemaphoreType.DMA,
        pltpu.SemaphoreType.DMA,
    ],
)
def _emb_sum(x0_hbm, x1_hbm, w0t_hbm, w1t_hbm, out_hbm,
             idx0_v, idx1_v,
             blk0a_v, blk0b_v, blk0c_v, blk0d_v, blk0e_v, blk0f_v,
             blk1a_v, blk1b_v, blk1c_v, blk1d_v, blk1e_v, blk1f_v,
             slab_v, sem0, sem1):
    wid = lax.axis_index("s") * _NC + lax.axis_index("c")
    base = wid * _BPW
    pltpu.sync_copy(x0_hbm.at[pl.ds(base, _BPW)], idx0_v)
    pltpu.sync_copy(x1_hbm.at[pl.ds(base, _BPW)], idx1_v)

    iota = lax.iota(jnp.int32, _L)
    dvs = [16 * c + iota for c in range(_D // _L)]
    blk0 = [blk0a_v, blk0b_v, blk0c_v, blk0d_v, blk0e_v, blk0f_v]
    blk1 = [blk1a_v, blk1b_v, blk1c_v, blk1d_v, blk1e_v, blk1f_v]
    _RING = 6

    _G = 2 * _L

    def group(g, carry):
        r0 = g * _G
        vecs0 = [idx0_v[pl.ds(r0, _L)], idx0_v[pl.ds(r0 + _L, _L)]]
        vecs1 = [idx1_v[pl.ds(r0, _L)], idx1_v[pl.ds(r0 + _L, _L)]]
        cbs0 = [jnp.minimum(lax.shift_right_logical(v, 7), _NCB - 1)
                for v in vecs0]
        cbs1 = [jnp.minimum(lax.shift_right_logical(v, 7), _NCB - 1)
                for v in vecs1]
        lanes0 = [v - cb * 128 for v, cb in zip(vecs0, cbs0)]
        lanes1 = [v - cb * 128 for v, cb in zip(vecs1, cbs1)]

        cps = [None] * _RING

        def fire(j):
            p = j % _RING
            q, r = divmod(j, _L)
            cp0 = pltpu.async_copy(
                w0t_hbm.at[:, pl.ds(cbs0[q][r] * 128, 128)], blk0[p], sem0)
            cp1 = pltpu.async_copy(
                w1t_hbm.at[:, pl.ds(cbs1[q][r] * 128, 128)], blk1[p], sem1)
            cps[p] = (cp0, cp1)

        for j in range(_RING - 1):
            fire(j)
        for j in range(_G):
            p = j % _RING
            cp0, cp1 = cps[p]
            cp0.wait()
            cp1.wait()
            if j + _RING - 1 < _G:
                fire(j + _RING - 1)
            q, r = divmod(j, _L)
            l0 = lanes0[q][r]
            l1 = lanes1[q][r]
            f0 = jnp.full((_L,), (l0 < 128).astype(jnp.float32))
            f1 = jnp.full((_L,), (l1 < 128).astype(jnp.float32))
            l0v = jnp.full((_L,), jnp.minimum(l0, 127))
            l1v = jnp.full((_L,), jnp.minimum(l1, 127))
            rv = jnp.full((_L,), lax.rem(g, 8) * _G + j)
            for c in range(_D // _L):
                e0 = plsc.load_gather(blk0[p], [dvs[c], l0v])
                e1 = plsc.load_gather(blk1[p], [dvs[c], l1v])
                plsc.store_scatter(slab_v, [dvs[c], rv], e0 * f0 + e1 * f1)
        @pl.when(lax.rem(g, 8) == 7)
        def _flush():
            half = lax.div(g, 8) * (_BPW // 2)
            pltpu.sync_copy(slab_v, out_hbm.at[:, pl.ds(base + half, _BPW // 2)])

        return carry

    lax.fori_loop(0, _BPW // _G, group, 0)


def kernel(x, W0, W1):
    x = x.astype(jnp.int32)
    x0 = x[:, 0]
    x1 = x[:, 1]
    out = _emb_sum(x0, x1, W0.T, W1.T).T
    tail0 = jnp.take(W0[_VMAIN:], jnp.clip(x0 - _VMAIN, 0, _V - _VMAIN - 1),
                     axis=0)
    tail1 = jnp.take(W1[_VMAIN:], jnp.clip(x1 - _VMAIN, 0, _V - _VMAIN - 1),
                     axis=0)
    out = out + jnp.where((x0 >= _VMAIN)[:, None], tail0, 0.0)
    out = out + jnp.where((x1 >= _VMAIN)[:, None], tail1, 0.0)
    return out
